# baseline (speedup 1.0000x reference)
import jax
import jax.numpy as jnp
from jax import lax
from jax.experimental import pallas as pl
from jax.experimental.pallas import tpu as pltpu

_S, _B, _MAXN, _MAXE, _ND, _H = 8, 32, 128, 2048, 8, 32
_EMB, _HEADS, _HD, _T = 32, 2, 16, 5
_ROWS = _S * _B
_N = _ROWS * _MAXN
_NEG = -1e30


def _lstm_body(xT_ref, wih_ref, whh_ref, bias_ref, out_ref):
    wih = wih_ref[...]
    whh = whh_ref[...]
    bias = bias_ref[...]
    bn = _B * _MAXN
    h = jnp.zeros((_H, bn), jnp.float32)
    c = jnp.zeros((_H, bn), jnp.float32)
    for s in range(_S):
        z = wih @ xT_ref[s] + whh @ h + bias
        i = jax.nn.sigmoid(z[0:_H])
        f = jax.nn.sigmoid(z[_H:2 * _H])
        g = jnp.tanh(z[2 * _H:3 * _H])
        o = jax.nn.sigmoid(z[3 * _H:4 * _H])
        c = f * c + i * g
        h = o * jnp.tanh(c)
        out_ref[s] = h


def _gat_layer(hT, a_srcT, a_dstT, wl, al, bl, relu):
    xl = wl @ hT
    xs = xl @ a_srcT
    xd = xl @ a_dstT
    u = xd + xs
    u = jnp.where(u > 0, u, 0.2 * u)
    eu = u * al
    e0 = jnp.sum(eu[0:_HD], axis=0, keepdims=True)
    e1 = jnp.sum(eu[_HD:2 * _HD], axis=0, keepdims=True)
    e2 = jnp.concatenate([e0, e1], axis=0)
    gm = jnp.max(e2, axis=1, keepdims=True)
    ex = jnp.exp(e2 - gm)
    ex_f = jnp.broadcast_to(
        ex[:, None, :], (_HEADS, _HD, _MAXE)).reshape(_HEADS * _HD, _MAXE)
    w = ex_f * xs
    cat = jnp.concatenate([ex, w], axis=0)
    both = lax.dot_general(cat, a_dstT, (((1,), (1,)), ((), ())))
    den = both[0:_HEADS]
    outT = both[_HEADS:]
    rec = 1.0 / (den + 1e-16)
    rec_f = jnp.broadcast_to(
        rec[:, None, :], (_HEADS, _HD, _MAXN)).reshape(_HEADS * _HD, _MAXN)
    hT = outT * rec_f + bl
    if relu:
        hT = jnp.maximum(hT, 0.0)
    return hT


_G = 8


def _gat_body(x_ref, ed_ref, reach_ref, w_ref, a_ref, b_ref, out_ref):
    iota_n = lax.broadcasted_iota(jnp.int32, (_MAXN, _MAXE), 0)
    hTs, a_s, a_d = [], [], []
    for g in range(_G):
        hTs.append(x_ref[0][:, g * _MAXN:(g + 1) * _MAXN])
        ed = ed_ref[0, g]
        a_s.append((ed[0:1].astype(jnp.int32) == iota_n).astype(jnp.float32))
        a_d.append((ed[1:2].astype(jnp.int32) == iota_n).astype(jnp.float32))
    for l in range(_T):
        for g in range(_G):
            hTs[g] = _gat_layer(hTs[g], a_s[g], a_d[g],
                                w_ref[l], a_ref[l], b_ref[l], l < _T - 1)
    pid = pl.program_id(0)
    bpg = _B // _G
    b0 = (pid % bpg) * _G
    for g in range(_G):
        out_ref[g, :, 0:_EMB] = hTs[g].T
        bval = (b0 + g).astype(jnp.float32)
        out_ref[g, :, _EMB:_EMB + 1] = jnp.full((_MAXN, 1), 1.0) * bval
        out_ref[g, :, _EMB + 1:_EMB + 2] = reach_ref[0, g]
        nnv = jnp.where(b0 + g < 2, 128.0, 0.0)
        out_ref[g, :, _EMB + 2:_EMB + 3] = jnp.full((_MAXN, 1), 1.0) * nnv


def kernel(state, W_ih, W_hh, b_ih, b_hh, gat_W, gat_a, gat_b):
    xT = state[:, :, :_ND * _MAXN].reshape(_S, _B * _MAXN, _ND).transpose(0, 2, 1)
    bias = (b_ih + b_hh)[:, None]
    lstm_out = pl.pallas_call(
        _lstm_body,
        out_shape=jax.ShapeDtypeStruct((_S, _H, _B * _MAXN), jnp.float32),
    )(xT, W_ih, W_hh, bias)

    edges = state[:, :, _ND * _MAXN:_ND * _MAXN + 2 * _MAXE].reshape(_S, _B, 2, _MAXE)
    Wt = gat_W.transpose(0, 1, 3, 2).reshape(_T, _HEADS * _HD, _EMB)
    at = gat_a.reshape(_T, _HEADS * _HD, 1)
    bt = gat_b[:, :, None]

    reach4 = state[:, :, _ND * _MAXN + 2 * _MAXE:
                   _ND * _MAXN + 2 * _MAXE + _MAXN].reshape(_S, _B, _MAXN, 1)
    bpg = _B // _G
    feat = pl.pallas_call(
        _gat_body,
        grid=(_ROWS // _G,),
        in_specs=[
            pl.BlockSpec((1, _H, _G * _MAXN), lambda r: (r // bpg, 0, r % bpg)),
            pl.BlockSpec((1, _G, 2, _MAXE), lambda r: (r // bpg, r % bpg, 0, 0)),
            pl.BlockSpec((1, _G, _MAXN, 1), lambda r: (r // bpg, r % bpg, 0, 0)),
            pl.BlockSpec((_T, _HEADS * _HD, _EMB), lambda r: (0, 0, 0)),
            pl.BlockSpec((_T, _HEADS * _HD, 1), lambda r: (0, 0, 0)),
            pl.BlockSpec((_T, _EMB, 1), lambda r: (0, 0, 0)),
        ],
        out_specs=pl.BlockSpec((_G, _MAXN, _EMB + 3), lambda r: (r, 0, 0)),
        out_shape=jax.ShapeDtypeStruct((_ROWS, _MAXN, _EMB + 3), jnp.float32),
        compiler_params=pltpu.CompilerParams(
            dimension_semantics=("arbitrary",)),
    )(lstm_out, edges, reach4, Wt, at, bt)

    splitval = _N // _S
    batch_ids = jnp.tile((jnp.arange(splitval) // _MAXN).astype(jnp.float32), _S)
    nn_first = jnp.concatenate([jnp.full((_ROWS,), float(_MAXN), jnp.float32),
                                jnp.zeros((_N - _ROWS,), jnp.float32)])[:splitval]
    num_nodes = jnp.tile(nn_first, _S)
    valid = jnp.stack([jnp.arange(_ROWS, dtype=jnp.int64) * _MAXN,
                       jnp.arange(_ROWS, dtype=jnp.int64) * _MAXN + _MAXN], axis=1)
    return (feat.reshape(_S, _N // _S, _EMB + 3), _N, valid, num_nodes)


# speedup vs baseline: 1.0184x; 1.0184x over previous
import jax
import jax.numpy as jnp
from jax import lax
from jax.experimental import pallas as pl
from jax.experimental.pallas import tpu as pltpu

_S, _B, _MAXN, _MAXE, _ND, _H = 8, 32, 128, 2048, 8, 32
_EMB, _HEADS, _HD, _T = 32, 2, 16, 5
_ROWS = _S * _B
_N = _ROWS * _MAXN
_NEG = -1e30


def _lstm_body(xT_ref, wih_ref, whh_ref, bias_ref, out_ref):
    wih = wih_ref[...]
    whh = whh_ref[...]
    bias = bias_ref[...]
    bn = _B * _MAXN
    h = jnp.zeros((_H, bn), jnp.float32)
    c = jnp.zeros((_H, bn), jnp.float32)
    for s in range(_S):
        z = wih @ xT_ref[s] + whh @ h + bias
        i = jax.nn.sigmoid(z[0:_H])
        f = jax.nn.sigmoid(z[_H:2 * _H])
        g = jnp.tanh(z[2 * _H:3 * _H])
        o = jax.nn.sigmoid(z[3 * _H:4 * _H])
        c = f * c + i * g
        h = o * jnp.tanh(c)
        out_ref[s] = h


def _stage_mm(hT, a_srcT, a_dstT, wl):
    xl = wl @ hT
    xs = xl @ a_srcT
    xd = xl @ a_dstT
    return xs, xd


def _stage_soft(xs, xd, a_dstT, al, bl, relu):
    u = xd + xs
    u = jnp.where(u > 0, u, 0.2 * u)
    eu = u * al
    e0 = jnp.sum(eu[0:_HD], axis=0, keepdims=True)
    e1 = jnp.sum(eu[_HD:2 * _HD], axis=0, keepdims=True)
    e2 = jnp.concatenate([e0, e1], axis=0)
    gm = jnp.max(e2, axis=1, keepdims=True)
    ex = jnp.exp(e2 - gm)
    ex_f = jnp.broadcast_to(
        ex[:, None, :], (_HEADS, _HD, _MAXE)).reshape(_HEADS * _HD, _MAXE)
    w = ex_f * xs
    cat = jnp.concatenate([ex, w], axis=0)
    both = lax.dot_general(cat, a_dstT, (((1,), (1,)), ((), ())))
    den = both[0:_HEADS]
    outT = both[_HEADS:]
    rec = 1.0 / (den + 1e-16)
    rec_f = jnp.broadcast_to(
        rec[:, None, :], (_HEADS, _HD, _MAXN)).reshape(_HEADS * _HD, _MAXN)
    hT = outT * rec_f + bl
    if relu:
        hT = jnp.maximum(hT, 0.0)
    return hT


_G = 8


def _gat_body(x_ref, ed_ref, reach_ref, w_ref, a_ref, b_ref, out_ref):
    iota_n = lax.broadcasted_iota(jnp.int32, (_MAXN, _MAXE), 0)
    hTs, a_s, a_d = [], [], []
    for g in range(_G):
        hTs.append(x_ref[0][:, g * _MAXN:(g + 1) * _MAXN])
        ed = ed_ref[0, g]
        a_s.append((ed[0:1].astype(jnp.int32) == iota_n).astype(jnp.float32))
        a_d.append((ed[1:2].astype(jnp.int32) == iota_n).astype(jnp.float32))
    for l in range(_T):
        relu = l < _T - 1
        mm = [None] * _G
        for g in range(_G):
            mm[g] = _stage_mm(hTs[g], a_s[g], a_d[g], w_ref[l])
            if g >= 1:
                hTs[g - 1] = _stage_soft(*mm[g - 1], a_d[g - 1],
                                         a_ref[l], b_ref[l], relu)
        hTs[_G - 1] = _stage_soft(*mm[_G - 1], a_d[_G - 1],
                                  a_ref[l], b_ref[l], relu)
    pid = pl.program_id(0)
    bpg = _B // _G
    b0 = (pid % bpg) * _G
    for g in range(_G):
        out_ref[g, :, 0:_EMB] = hTs[g].T
        bval = (b0 + g).astype(jnp.float32)
        out_ref[g, :, _EMB:_EMB + 1] = jnp.full((_MAXN, 1), 1.0) * bval
        out_ref[g, :, _EMB + 1:_EMB + 2] = reach_ref[0, g]
        nnv = jnp.where(b0 + g < 2, 128.0, 0.0)
        out_ref[g, :, _EMB + 2:_EMB + 3] = jnp.full((_MAXN, 1), 1.0) * nnv


def kernel(state, W_ih, W_hh, b_ih, b_hh, gat_W, gat_a, gat_b):
    xT = state[:, :, :_ND * _MAXN].reshape(_S, _B * _MAXN, _ND).transpose(0, 2, 1)
    bias = (b_ih + b_hh)[:, None]
    lstm_out = pl.pallas_call(
        _lstm_body,
        out_shape=jax.ShapeDtypeStruct((_S, _H, _B * _MAXN), jnp.float32),
    )(xT, W_ih, W_hh, bias)

    edges = state[:, :, _ND * _MAXN:_ND * _MAXN + 2 * _MAXE].reshape(_S, _B, 2, _MAXE)
    Wt = gat_W.transpose(0, 1, 3, 2).reshape(_T, _HEADS * _HD, _EMB)
    at = gat_a.reshape(_T, _HEADS * _HD, 1)
    bt = gat_b[:, :, None]

    reach4 = state[:, :, _ND * _MAXN + 2 * _MAXE:
                   _ND * _MAXN + 2 * _MAXE + _MAXN].reshape(_S, _B, _MAXN, 1)
    bpg = _B // _G
    feat = pl.pallas_call(
        _gat_body,
        grid=(_ROWS // _G,),
        in_specs=[
            pl.BlockSpec((1, _H, _G * _MAXN), lambda r: (r // bpg, 0, r % bpg)),
            pl.BlockSpec((1, _G, 2, _MAXE), lambda r: (r // bpg, r % bpg, 0, 0)),
            pl.BlockSpec((1, _G, _MAXN, 1), lambda r: (r // bpg, r % bpg, 0, 0)),
            pl.BlockSpec((_T, _HEADS * _HD, _EMB), lambda r: (0, 0, 0)),
            pl.BlockSpec((_T, _HEADS * _HD, 1), lambda r: (0, 0, 0)),
            pl.BlockSpec((_T, _EMB, 1), lambda r: (0, 0, 0)),
        ],
        out_specs=pl.BlockSpec((_G, _MAXN, _EMB + 3), lambda r: (r, 0, 0)),
        out_shape=jax.ShapeDtypeStruct((_ROWS, _MAXN, _EMB + 3), jnp.float32),
        compiler_params=pltpu.CompilerParams(
            dimension_semantics=("arbitrary",)),
    )(lstm_out, edges, reach4, Wt, at, bt)

    splitval = _N // _S
    batch_ids = jnp.tile((jnp.arange(splitval) // _MAXN).astype(jnp.float32), _S)
    nn_first = jnp.concatenate([jnp.full((_ROWS,), float(_MAXN), jnp.float32),
                                jnp.zeros((_N - _ROWS,), jnp.float32)])[:splitval]
    num_nodes = jnp.tile(nn_first, _S)
    valid = jnp.stack([jnp.arange(_ROWS, dtype=jnp.int64) * _MAXN,
                       jnp.arange(_ROWS, dtype=jnp.int64) * _MAXN + _MAXN], axis=1)
    return (feat.reshape(_S, _N // _S, _EMB + 3), _N, valid, num_nodes)
